# baseline (speedup 1.0000x reference)
import functools

import jax
import jax.numpy as jnp
from jax import lax
from jax.experimental import pallas as pl
from jax.experimental.pallas import tpu as pltpu
from jax.experimental.pallas import tpu_sc as plsc

B, N, D, K = 4, 2048, 128, 16
L = 16
NC, NS = 2, 16
NW = NC * NS
QPT = (B * N) // NW
TPB = N // QPT
NBLK = N // L
NGRP = QPT // L
IDX_ROWS = (QPT * K) // 128
BIG = 1e30
IMAX = 2**31 - 1
NSTRIPE = 16
BPS = NBLK // NSTRIPE
CAP = 256


def _body(cen_hbm, fr9_hbm, attr_hbm, out_c_hbm, out_a_hbm,
          cxv, cyv, czv, penv, frv, idxbuf, coordbuf, dbuf, mbuf, jbucket,
          gbuf, gsem, wsem):
    wid = lax.axis_index("s") * NC + lax.axis_index("c")
    batch = wid // TPB
    q0 = (wid % TPB) * QPT

    pltpu.sync_copy(cen_hbm.at[batch, 0], cxv)
    pltpu.sync_copy(cen_hbm.at[batch, 1], cyv)
    pltpu.sync_copy(cen_hbm.at[batch, 2], czv)
    pltpu.sync_copy(cen_hbm.at[batch, 3], penv)
    pltpu.sync_copy(fr9_hbm.at[batch, :, pl.ds(q0, QPT)], frv)

    lane = lax.iota(jnp.int32, L)

    def group_body(g, _):
        qsel = q0 + g * L + lane
        qx = plsc.load_gather(cxv, [qsel])
        qy = plsc.load_gather(cyv, [qsel])
        qz = plsc.load_gather(czv, [qsel])

        rots = [(lane + r) & (L - 1) for r in range(L)]
        zero16 = jnp.zeros((L,), jnp.int32)

        def quad_body(qd, _):
            qb = []
            for qi in range(4):
                qsel1 = zero16 + (q0 + g * L + qd * 4 + qi)
                qb.append((plsc.load_gather(cxv, [qsel1]),
                           plsc.load_gather(cyv, [qsel1]),
                           plsc.load_gather(czv, [qsel1])))

            minit = tuple(jnp.full((L,), BIG, jnp.float32) for _ in range(4))

            @plsc.parallel_loop(0, NBLK, carry=minit)
            def mq(bb, m_c):
                base = bb * L
                cxb = cxv[pl.ds(base, L)]
                cyb = cyv[pl.ds(base, L)]
                czb = czv[pl.ds(base, L)]
                ppb = penv[pl.ds(base, L)]
                m_n = list(m_c)
                for qi in range(4):
                    bx, by, bz = qb[qi]
                    dx = bx - cxb
                    dy = by - cyb
                    dz = bz - czb
                    d = ((dx * dx + dy * dy) + dz * dz) + ppb
                    plsc.store_scatter(
                        dbuf, [(qd * 4 + qi) * N + base + lane], d)
                    m_n[qi] = jnp.minimum(m_n[qi], d)
                return tuple(m_n)

            for qi in range(4):
                plsc.store_scatter(mbuf, [(qd * 4 + qi) * L + lane], mq[qi])
            return 0

        lax.fori_loop(0, 4, quad_body, 0, unroll=False)

        u = jnp.full((L,), -BIG, jnp.float32)
        for c in range(L):
            u = jnp.maximum(u, plsc.load_gather(mbuf, [lane * L + c]))

        laneN = lane * N

        laneCAP = lane * CAP

        @plsc.parallel_loop(0, NBLK, carry=jnp.zeros((L,), jnp.int32))
        def cnt(bb, cnt_c):
            lb = laneN + bb * L
            addrs = [lb + rots[r] for r in range(L)]
            keeps = [plsc.load_gather(dbuf, [a]) <= u for a in addrs]
            prefs = []
            for r in range(L):
                prefs.append(cnt_c)
                cnt_c = cnt_c + keeps[r].astype(jnp.int32)
            for r in range(L):
                pos = laneCAP + jnp.minimum(prefs[r], CAP - 1)
                plsc.store_scatter(jbucket, [pos], addrs[r], mask=keeps[r])
            return cnt_c
        maxcnt = jnp.minimum(jnp.max(cnt), CAP)

        init = (tuple(jnp.full((L,), BIG, jnp.float32) for _ in range(K))
                + tuple(jnp.full((L,), IMAX, jnp.int32) for _ in range(K)))

        def pb_body(tb, carry):
            bd = list(carry[:K])
            bi = list(carry[K:])
            cands = []
            for uu in range(4):
                t = tb * 4 + uu
                valid = t < cnt
                addr = plsc.load_gather(jbucket, [lane * CAP + t]) & (N * L - 1)
                cd = plsc.load_gather(dbuf, [addr])
                ci = addr & (N - 1)
                cands.append((jnp.where(valid, cd, BIG),
                              jnp.where(valid, ci, IMAX)))
            for cd, ci in cands:
                for t2 in range(K):
                    lt = (cd < bd[t2]) | ((cd == bd[t2]) & (ci < bi[t2]))
                    td = jnp.where(lt, cd, bd[t2])
                    ti = jnp.where(lt, ci, bi[t2])
                    cd = jnp.where(lt, bd[t2], cd)
                    ci = jnp.where(lt, bi[t2], ci)
                    bd[t2] = td
                    bi[t2] = ti
            return tuple(bd) + tuple(bi)

        final = lax.fori_loop(0, (maxcnt + 3) // 4, pb_body, init,
                              unroll=False)
        bd = final[:K]
        bi = final[K:]

        f = [plsc.load_gather(frv, [jnp.full((L,), k9, jnp.int32),
                                    g * L + lane]) for k9 in range(9)]
        for r in range(K):
            nb = bi[r]
            p = g * (L * K) + lane * K + r
            plsc.store_scatter(idxbuf, [(p >> 7) & 3, p & 127], nb + batch * N)
            gx = plsc.load_gather(cxv, [nb])
            gy = plsc.load_gather(cyv, [nb])
            gz = plsc.load_gather(czv, [nb])
            ddx = gx - qx
            ddy = gy - qy
            ddz = gz - qz
            cbase = g * (L * K * 3) + lane * (K * 3) + r * 3
            for m in range(3):
                cm = ddx * f[3 * m] + ddy * f[3 * m + 1] + ddz * f[3 * m + 2]
                plsc.store_scatter(coordbuf, [cbase + m], cm)

        GR = L * K
        pg = g & 1

        @pl.when(g >= 2)
        def _wait_prev_wb():
            pltpu.make_async_copy(gbuf.at[pg], out_a_hbm.at[pl.ds(0, GR)],
                                  wsem.at[pg]).wait()

        pltpu.async_copy(attr_hbm.at[idxbuf.at[(2 * g) & 3]],
                         gbuf.at[pg, pl.ds(0, 128)], gsem.at[pg])
        pltpu.async_copy(attr_hbm.at[idxbuf.at[(2 * g + 1) & 3]],
                         gbuf.at[pg, pl.ds(128, 128)], gsem.at[pg])

        @pl.when(g >= 1)
        def _drain_prev_gather_and_writeback():
            po = 1 - pg
            pltpu.make_async_copy(attr_hbm.at[pl.ds(0, GR)], gbuf.at[po],
                                  gsem.at[po]).wait()
            pltpu.async_copy(
                gbuf.at[po],
                out_a_hbm.at[pl.ds(wid * (QPT * K) + (g - 1) * GR, GR)],
                wsem.at[po])
        return 0

    lax.fori_loop(0, NGRP, group_body, 0, unroll=False)

    GR = L * K
    pltpu.make_async_copy(attr_hbm.at[pl.ds(0, GR)], gbuf.at[1],
                          gsem.at[1]).wait()
    pltpu.async_copy(gbuf.at[1],
                     out_a_hbm.at[pl.ds(wid * (QPT * K) + (NGRP - 1) * GR,
                                        GR)], wsem.at[1])
    pltpu.sync_copy(coordbuf, out_c_hbm.at[pl.ds(wid * (QPT * K * 3),
                                                 QPT * K * 3)])
    pltpu.make_async_copy(gbuf.at[0], out_a_hbm.at[pl.ds(0, GR)],
                          wsem.at[0]).wait()
    pltpu.make_async_copy(gbuf.at[1], out_a_hbm.at[pl.ds(0, GR)],
                          wsem.at[1]).wait()


@jax.jit
def _run(cen, fr9, attr_flat):
    mesh = plsc.VectorSubcoreMesh(core_axis_name="c", subcore_axis_name="s",
                                  num_cores=NC, num_subcores=NS)
    return pl.kernel(
        _body,
        out_type=[
            jax.ShapeDtypeStruct((B * N * K * 3,), jnp.float32),
            jax.ShapeDtypeStruct((B * N * K, D), jnp.float32),
        ],
        mesh=mesh,
        compiler_params=pltpu.CompilerParams(needs_layout_passes=False),
        scratch_types=[
            pltpu.VMEM((N,), jnp.float32),
            pltpu.VMEM((N,), jnp.float32),
            pltpu.VMEM((N,), jnp.float32),
            pltpu.VMEM((N,), jnp.float32),
            pltpu.VMEM((9, QPT), jnp.float32),
            pltpu.VMEM((4, 128), jnp.int32),
            pltpu.VMEM((QPT * K * 3,), jnp.float32),
            pltpu.VMEM((N * L,), jnp.float32),
            pltpu.VMEM((L * L,), jnp.float32),
            pltpu.VMEM((L * CAP,), jnp.int32),
            pltpu.VMEM((2, L * K, D), jnp.float32),
            pltpu.SemaphoreType.DMA((2,)),
            pltpu.SemaphoreType.DMA((2,)),
        ],
    )(cen, fr9, attr_flat)


def kernel(frame, attributes, mask):
    centers = frame[:, :, 0, :]
    pen = 2000.0 * (1.0 - mask[0][:, :, 1])
    cen = jnp.concatenate(
        [jnp.moveaxis(centers, -1, 1), pen[:, None, :]], axis=1)
    fr9 = jnp.moveaxis(frame[:, :, 1:4, :].reshape(B, N, 9), -1, 1)
    attr_flat = attributes.reshape(B * N, D)
    coords, attrs = _run(cen, fr9, attr_flat)
    return (coords.reshape(B, N, K, 3), attrs.reshape(B, N, K, D))


# speedup vs baseline: 1.0134x; 1.0134x over previous
import functools

import jax
import jax.numpy as jnp
from jax import lax
from jax.experimental import pallas as pl
from jax.experimental.pallas import tpu as pltpu
from jax.experimental.pallas import tpu_sc as plsc

B, N, D, K = 4, 2048, 128, 16
L = 16
NC, NS = 2, 16
NW = NC * NS
QPT = (B * N) // NW
TPB = N // QPT
NBLK = N // L
NGRP = QPT // L
IDX_ROWS = (QPT * K) // 128
BIG = 1e30
IMAX = 2**31 - 1
NSTRIPE = 16
BPS = NBLK // NSTRIPE
CAP = 256


def _body(cen_hbm, fr9_hbm, attr_hbm, out_c_hbm, out_a_hbm,
          cxv, cyv, czv, penv, frv, idxbuf, coordbuf, dbuf, mbuf, jbucket,
          gbuf, gsem, wsem):
    wid = lax.axis_index("s") * NC + lax.axis_index("c")
    batch = wid // TPB
    q0 = (wid % TPB) * QPT

    pltpu.sync_copy(cen_hbm.at[batch, 0], cxv)
    pltpu.sync_copy(cen_hbm.at[batch, 1], cyv)
    pltpu.sync_copy(cen_hbm.at[batch, 2], czv)
    pltpu.sync_copy(cen_hbm.at[batch, 3], penv)
    pltpu.sync_copy(fr9_hbm.at[batch, :, pl.ds(q0, QPT)], frv)

    lane = lax.iota(jnp.int32, L)

    def group_body(g, _):
        qsel = q0 + g * L + lane
        qx = plsc.load_gather(cxv, [qsel])
        qy = plsc.load_gather(cyv, [qsel])
        qz = plsc.load_gather(czv, [qsel])

        rots = [(lane + r) & (L - 1) for r in range(L)]
        zero16 = jnp.zeros((L,), jnp.int32)

        def quad_body(qd, _):
            qb = []
            for qi in range(4):
                qsel1 = zero16 + (q0 + g * L + qd * 4 + qi)
                qb.append((plsc.load_gather(cxv, [qsel1]),
                           plsc.load_gather(cyv, [qsel1]),
                           plsc.load_gather(czv, [qsel1])))

            minit = tuple(jnp.full((L,), BIG, jnp.float32) for _ in range(4))

            @plsc.parallel_loop(0, NBLK, carry=minit)
            def mq(bb, m_c):
                base = bb * L
                cxb = cxv[pl.ds(base, L)]
                cyb = cyv[pl.ds(base, L)]
                czb = czv[pl.ds(base, L)]
                ppb = penv[pl.ds(base, L)]
                m_n = list(m_c)
                for qi in range(4):
                    bx, by, bz = qb[qi]
                    dx = bx - cxb
                    dy = by - cyb
                    dz = bz - czb
                    d = ((dx * dx + dy * dy) + dz * dz) + ppb
                    plsc.store_scatter(
                        dbuf, [(qd * 4 + qi) * N + base + lane], d)
                    m_n[qi] = jnp.minimum(m_n[qi], d)
                return tuple(m_n)

            for qi in range(4):
                plsc.store_scatter(mbuf, [(qd * 4 + qi) * L + lane], mq[qi])
            return 0

        lax.fori_loop(0, 4, quad_body, 0, unroll=False)

        u = jnp.full((L,), -BIG, jnp.float32)
        for c in range(L):
            u = jnp.maximum(u, plsc.load_gather(mbuf, [lane * L + c]))

        laneN = lane * N

        laneCAP = lane * CAP

        @plsc.parallel_loop(0, NBLK, carry=jnp.zeros((L,), jnp.int32))
        def cnt(bb, cnt_c):
            lb = laneN + bb * L
            addrs = [lb + rots[r] for r in range(L)]
            keeps = [plsc.load_gather(dbuf, [a]) <= u for a in addrs]
            prefs = []
            for r in range(L):
                prefs.append(cnt_c)
                cnt_c = cnt_c + keeps[r].astype(jnp.int32)
            for r in range(L):
                pos = laneCAP + jnp.minimum(prefs[r], CAP - 1)
                plsc.store_scatter(jbucket, [pos], addrs[r], mask=keeps[r])
            return cnt_c
        maxcnt = jnp.minimum(jnp.max(cnt), CAP)

        init = (tuple(jnp.full((L,), BIG, jnp.float32) for _ in range(K))
                + tuple(jnp.full((L,), IMAX, jnp.int32) for _ in range(K)))

        def pb_body(tb, carry):
            bd = list(carry[:K])
            bi = list(carry[K:])
            cands = []
            for uu in range(8):
                t = tb * 8 + uu
                valid = t < cnt
                addr = plsc.load_gather(jbucket, [lane * CAP + t]) & (N * L - 1)
                cd = plsc.load_gather(dbuf, [addr])
                ci = addr & (N - 1)
                cands.append((jnp.where(valid, cd, BIG),
                              jnp.where(valid, ci, IMAX)))
            for cd, ci in cands:
                for t2 in range(K):
                    lt = (cd < bd[t2]) | ((cd == bd[t2]) & (ci < bi[t2]))
                    td = jnp.where(lt, cd, bd[t2])
                    ti = jnp.where(lt, ci, bi[t2])
                    cd = jnp.where(lt, bd[t2], cd)
                    ci = jnp.where(lt, bi[t2], ci)
                    bd[t2] = td
                    bi[t2] = ti
            return tuple(bd) + tuple(bi)

        final = lax.fori_loop(0, (maxcnt + 7) // 8, pb_body, init,
                              unroll=False)
        bd = final[:K]
        bi = final[K:]

        f = [plsc.load_gather(frv, [jnp.full((L,), k9, jnp.int32),
                                    g * L + lane]) for k9 in range(9)]
        for r in range(K):
            nb = bi[r]
            p = g * (L * K) + lane * K + r
            plsc.store_scatter(idxbuf, [(p >> 7) & 3, p & 127], nb + batch * N)
            gx = plsc.load_gather(cxv, [nb])
            gy = plsc.load_gather(cyv, [nb])
            gz = plsc.load_gather(czv, [nb])
            ddx = gx - qx
            ddy = gy - qy
            ddz = gz - qz
            cbase = g * (L * K * 3) + lane * (K * 3) + r * 3
            for m in range(3):
                cm = ddx * f[3 * m] + ddy * f[3 * m + 1] + ddz * f[3 * m + 2]
                plsc.store_scatter(coordbuf, [cbase + m], cm)

        GR = L * K
        pg = g & 1

        @pl.when(g >= 2)
        def _wait_prev_wb():
            pltpu.make_async_copy(gbuf.at[pg], out_a_hbm.at[pl.ds(0, GR)],
                                  wsem.at[pg]).wait()

        pltpu.async_copy(attr_hbm.at[idxbuf.at[(2 * g) & 3]],
                         gbuf.at[pg, pl.ds(0, 128)], gsem.at[pg])
        pltpu.async_copy(attr_hbm.at[idxbuf.at[(2 * g + 1) & 3]],
                         gbuf.at[pg, pl.ds(128, 128)], gsem.at[pg])

        @pl.when(g >= 1)
        def _drain_prev_gather_and_writeback():
            po = 1 - pg
            pltpu.make_async_copy(attr_hbm.at[pl.ds(0, GR)], gbuf.at[po],
                                  gsem.at[po]).wait()
            pltpu.async_copy(
                gbuf.at[po],
                out_a_hbm.at[pl.ds(wid * (QPT * K) + (g - 1) * GR, GR)],
                wsem.at[po])
        return 0

    lax.fori_loop(0, NGRP, group_body, 0, unroll=False)

    GR = L * K
    pltpu.make_async_copy(attr_hbm.at[pl.ds(0, GR)], gbuf.at[1],
                          gsem.at[1]).wait()
    pltpu.async_copy(gbuf.at[1],
                     out_a_hbm.at[pl.ds(wid * (QPT * K) + (NGRP - 1) * GR,
                                        GR)], wsem.at[1])
    pltpu.sync_copy(coordbuf, out_c_hbm.at[pl.ds(wid * (QPT * K * 3),
                                                 QPT * K * 3)])
    pltpu.make_async_copy(gbuf.at[0], out_a_hbm.at[pl.ds(0, GR)],
                          wsem.at[0]).wait()
    pltpu.make_async_copy(gbuf.at[1], out_a_hbm.at[pl.ds(0, GR)],
                          wsem.at[1]).wait()


@jax.jit
def _run(cen, fr9, attr_flat):
    mesh = plsc.VectorSubcoreMesh(core_axis_name="c", subcore_axis_name="s",
                                  num_cores=NC, num_subcores=NS)
    return pl.kernel(
        _body,
        out_type=[
            jax.ShapeDtypeStruct((B * N * K * 3,), jnp.float32),
            jax.ShapeDtypeStruct((B * N * K, D), jnp.float32),
        ],
        mesh=mesh,
        compiler_params=pltpu.CompilerParams(needs_layout_passes=False),
        scratch_types=[
            pltpu.VMEM((N,), jnp.float32),
            pltpu.VMEM((N,), jnp.float32),
            pltpu.VMEM((N,), jnp.float32),
            pltpu.VMEM((N,), jnp.float32),
            pltpu.VMEM((9, QPT), jnp.float32),
            pltpu.VMEM((4, 128), jnp.int32),
            pltpu.VMEM((QPT * K * 3,), jnp.float32),
            pltpu.VMEM((N * L,), jnp.float32),
            pltpu.VMEM((L * L,), jnp.float32),
            pltpu.VMEM((L * CAP,), jnp.int32),
            pltpu.VMEM((2, L * K, D), jnp.float32),
            pltpu.SemaphoreType.DMA((2,)),
            pltpu.SemaphoreType.DMA((2,)),
        ],
    )(cen, fr9, attr_flat)


def kernel(frame, attributes, mask):
    centers = frame[:, :, 0, :]
    pen = 2000.0 * (1.0 - mask[0][:, :, 1])
    cen = jnp.concatenate(
        [jnp.moveaxis(centers, -1, 1), pen[:, None, :]], axis=1)
    fr9 = jnp.moveaxis(frame[:, :, 1:4, :].reshape(B, N, 9), -1, 1)
    attr_flat = attributes.reshape(B * N, D)
    coords, attrs = _run(cen, fr9, attr_flat)
    return (coords.reshape(B, N, K, 3), attrs.reshape(B, N, K, D))
